# baseline (speedup 1.0000x reference)
import functools

import jax
import jax.numpy as jnp
from jax.experimental import pallas as pl
from jax.experimental.pallas import tpu as pltpu

CUTOFFS = [2000, 10000, 50000]
SHORTLIST = 2000
NEG_INF = float("-inf")
TB = 256


def _cluster_of(t):
    return ((t >= CUTOFFS[0]).astype(jnp.int32)
            + (t >= CUTOFFS[1]).astype(jnp.int32)
            + (t >= CUTOFFS[2]).astype(jnp.int32))


def _prep1_body(t_ref, pos_ref, cnt_ref, aoff_ref, *, n_tokens):
    tb = pl.program_id(0)
    t_all = t_ref[...]
    cl_all = _cluster_of(t_all)
    cl_row = jnp.transpose(cl_all)
    row0 = tb * TB
    cl_blk = _cluster_of(t_ref[pl.ds(row0, TB), :])
    cols = jax.lax.broadcasted_iota(jnp.int32, (TB, n_tokens), 1)
    rows = row0 + jax.lax.broadcasted_iota(jnp.int32, (TB, n_tokens), 0)
    rank = jnp.sum(jnp.where((cl_row == cl_blk) & (cols < rows), 1, 0),
                   axis=1, keepdims=True)
    cidx = jax.lax.broadcasted_iota(jnp.int32, (8, n_tokens), 0)
    cnt = jnp.sum((cl_row == cidx).astype(jnp.int32), axis=1, keepdims=True)
    rup = ((cnt + (TB - 1)) // TB) * TB
    c_r = jax.lax.broadcasted_iota(jnp.int32, (8, 8), 0)
    c_c = jax.lax.broadcasted_iota(jnp.int32, (8, 8), 1)
    aoff = jnp.sum(jnp.where(c_c < c_r, jnp.transpose(rup), 0), axis=1,
                   keepdims=True)
    cl8 = jax.lax.broadcasted_iota(jnp.int32, (TB, 8), 1)
    my_off = jnp.sum(jnp.where(cl_blk == cl8, jnp.transpose(aoff), 0),
                     axis=1, keepdims=True)
    pos_ref[...] = my_off + rank

    @pl.when(tb == 0)
    def _counts():
        cnt_ref[...] = cnt
        aoff_ref[...] = aoff


def _prep2_body(x_ref, w11_ref, w12_ref, w13_ref, t_ref, pos_ref,
                h1_ref, h2_ref, h3_ref, ts_ref, hid1_ref, hid2_ref, hid3_ref,
                *, n_tokens):
    tb = pl.program_id(0)

    @pl.when(tb == 0)
    def _hidden():
        xb = x_ref[...].astype(jnp.bfloat16)
        for w_ref, hid_ref in ((w11_ref, hid1_ref), (w12_ref, hid2_ref),
                               (w13_ref, hid3_ref)):
            hid_ref[...] = jax.lax.dot_general(
                xb, w_ref[...].astype(jnp.bfloat16), (((1,), (1,)), ((), ())),
                preferred_element_type=jnp.float32).astype(jnp.bfloat16)

    row0 = tb * TB
    p_rows = row0 + jax.lax.broadcasted_iota(jnp.int32, (TB, n_tokens), 0)
    pos_row = jnp.transpose(pos_ref[...])
    s_blk = (pos_row == p_rows)
    s_bf = s_blk.astype(jnp.bfloat16)
    h1_ref[...] = jax.lax.dot_general(
        s_bf, hid1_ref[...], (((1,), (0,)), ((), ())),
        preferred_element_type=jnp.float32).astype(jnp.bfloat16)
    h2_ref[...] = jax.lax.dot_general(
        s_bf, hid2_ref[...], (((1,), (0,)), ((), ())),
        preferred_element_type=jnp.float32).astype(jnp.bfloat16)
    h3_ref[...] = jax.lax.dot_general(
        s_bf, hid3_ref[...], (((1,), (0,)), ((), ())),
        preferred_element_type=jnp.float32).astype(jnp.bfloat16)
    t_row = jnp.transpose(t_ref[...])
    ts_ref[...] = jnp.sum(jnp.where(s_blk, t_row, 0), axis=1, keepdims=True)


def _tail_body(oc_ref, h_ref, w2_ref, ts_ref, out_ref, m_ref, s_ref, tg_ref,
               *, low, n_cols, blk, n_blk, n_tb):
    cb = pl.program_id(0)
    off = oc_ref[0]
    cnt = oc_ref[1]

    @pl.when(cb == 0)
    def _init():
        m_ref[...] = jnp.full(m_ref.shape, NEG_INF, jnp.float32)
        s_ref[...] = jnp.zeros(s_ref.shape, jnp.float32)
        tg_ref[...] = jnp.zeros(tg_ref.shape, jnp.float32)

    def _update(masked):
        w2b = w2_ref[...].astype(jnp.bfloat16)
        for j in range(n_tb):
            row0 = j * TB
            active = (row0 < off + cnt) & (row0 + TB > off)

            @pl.when(active)
            def _step(row0=row0):
                hid = h_ref[row0:row0 + TB, :]
                logits = jax.lax.dot_general(
                    hid, w2b, (((1,), (1,)), ((), ())),
                    preferred_element_type=jnp.float32)
                cols = cb * blk + jax.lax.broadcasted_iota(
                    jnp.int32, logits.shape, 1)
                if masked:
                    lm = jnp.where(cols < n_cols, logits, NEG_INF)
                else:
                    lm = logits
                bm = jnp.max(lm, axis=1, keepdims=True)
                m_old = m_ref[row0:row0 + TB, :]
                m_new = jnp.maximum(m_old, bm)
                s_ref[row0:row0 + TB, :] = (
                    s_ref[row0:row0 + TB, :] * jnp.exp(m_old - m_new)
                    + jnp.sum(jnp.exp(lm - m_new), axis=1, keepdims=True))
                m_ref[row0:row0 + TB, :] = m_new
                rel = jnp.clip(ts_ref[row0:row0 + TB, :] - low, 0, n_cols - 1)
                tg_ref[row0:row0 + TB, :] = tg_ref[row0:row0 + TB, :] + jnp.sum(
                    jnp.where(cols == rel, lm, 0.0), axis=1, keepdims=True)

    @pl.when(cb < n_blk - 1)
    def _full_blocks():
        _update(masked=False)

    @pl.when(cb == n_blk - 1)
    def _last_block():
        _update(masked=True)
        rows = jax.lax.broadcasted_iota(jnp.int32, (m_ref.shape[0], 1), 0)
        valid = (rows >= off) & (rows < off + cnt)
        val = tg_ref[...] - m_ref[...] - jnp.log(s_ref[...])
        out_ref[...] = jnp.where(valid, val, 0.0)


def _tail_logprob(offcnt, h_sorted, w2, ts, low, blk):
    T = h_sorted.shape[0]
    n_cols, h = w2.shape
    n_blk = pl.cdiv(n_cols, blk)
    n_tb = T // TB
    body = functools.partial(_tail_body, low=low, n_cols=n_cols, blk=blk,
                             n_blk=n_blk, n_tb=n_tb)
    grid_spec = pltpu.PrefetchScalarGridSpec(
        num_scalar_prefetch=1,
        grid=(n_blk,),
        in_specs=[
            pl.BlockSpec((T, h), lambda cb, oc: (0, 0)),
            pl.BlockSpec((blk, h), lambda cb, oc: (cb, 0)),
            pl.BlockSpec((T, 1), lambda cb, oc: (0, 0)),
        ],
        out_specs=pl.BlockSpec((T, 1), lambda cb, oc: (0, 0)),
        scratch_shapes=[
            pltpu.VMEM((T, 1), jnp.float32),
            pltpu.VMEM((T, 1), jnp.float32),
            pltpu.VMEM((T, 1), jnp.float32),
        ],
    )
    return pl.pallas_call(
        body,
        grid_spec=grid_spec,
        out_shape=jax.ShapeDtypeStruct((T, 1), jnp.float32),
    )(offcnt, h_sorted, w2, ts)


def _head_body(x_ref, hw_ref, t_ref, pos_ref, l1_ref, l2_ref, l3_ref, out_ref,
               loss_ref, acc_ref, *, n_tb, n_tokens):
    tb = pl.program_id(0)
    logits = jax.lax.dot_general(
        x_ref[...].astype(jnp.bfloat16), hw_ref[...].astype(jnp.bfloat16),
        (((1,), (1,)), ((), ())),
        preferred_element_type=jnp.float32)
    cols = jax.lax.broadcasted_iota(jnp.int32, logits.shape, 1)
    m = jnp.max(logits, axis=1, keepdims=True)
    s = jnp.sum(jnp.exp(logits - m), axis=1, keepdims=True)
    t = t_ref[...]
    cl = _cluster_of(t)
    gidx = jnp.where(cl == 0, t, SHORTLIST + cl - 1)
    ht = (jnp.sum(jnp.where(cols == gidx, logits, 0.0), axis=1, keepdims=True)
          - m - jnp.log(s))
    lsum = l1_ref[...] + l2_ref[...] + l3_ref[...]
    p_rows = jax.lax.broadcasted_iota(jnp.int32, (lsum.shape[0], TB), 0)
    s_blk = (jnp.transpose(pos_ref[...]) == p_rows).astype(jnp.float32)
    local = jax.lax.dot_general(
        s_blk, lsum, (((0,), (0,)), ((), ())),
        preferred_element_type=jnp.float32)
    out = local + ht
    out_ref[...] = out

    @pl.when(tb == 0)
    def _z():
        acc_ref[0] = 0.0

    acc_ref[0] += jnp.sum(-out) / n_tokens

    @pl.when(tb == n_tb - 1)
    def _w():
        loss_ref[...] = jnp.full((1, 1), acc_ref[0], jnp.float32)


def kernel(myinput, target, head_W, W1_1, W2_1, W1_2, W2_2, W1_3, W2_3):
    x = myinput
    T, F = x.shape
    t2 = target.astype(jnp.int32).reshape(T, 1)
    n_tb = T // TB

    TS = T + 3 * TB
    n_ts = TS // TB
    pos, cnt8, aoff8 = pl.pallas_call(
        functools.partial(_prep1_body, n_tokens=T),
        grid=(n_tb,),
        in_specs=[pl.BlockSpec((T, 1), lambda tb: (0, 0))],
        out_specs=[
            pl.BlockSpec((TB, 1), lambda tb: (tb, 0)),
            pl.BlockSpec((8, 1), lambda tb: (0, 0)),
            pl.BlockSpec((8, 1), lambda tb: (0, 0)),
        ],
        out_shape=[
            jax.ShapeDtypeStruct((T, 1), jnp.int32),
            jax.ShapeDtypeStruct((8, 1), jnp.int32),
            jax.ShapeDtypeStruct((8, 1), jnp.int32),
        ],
    )(t2)

    hs = [W1_1.shape[0], W1_2.shape[0], W1_3.shape[0]]
    h1s, h2s, h3s, ts = pl.pallas_call(
        functools.partial(_prep2_body, n_tokens=T),
        grid=(n_ts,),
        in_specs=[
            pl.BlockSpec((T, F), lambda tb: (0, 0)),
            pl.BlockSpec(W1_1.shape, lambda tb: (0, 0)),
            pl.BlockSpec(W1_2.shape, lambda tb: (0, 0)),
            pl.BlockSpec(W1_3.shape, lambda tb: (0, 0)),
            pl.BlockSpec((T, 1), lambda tb: (0, 0)),
            pl.BlockSpec((T, 1), lambda tb: (0, 0)),
        ],
        out_specs=[
            pl.BlockSpec((TB, hs[0]), lambda tb: (tb, 0)),
            pl.BlockSpec((TB, hs[1]), lambda tb: (tb, 0)),
            pl.BlockSpec((TB, hs[2]), lambda tb: (tb, 0)),
            pl.BlockSpec((TB, 1), lambda tb: (tb, 0)),
        ],
        out_shape=[
            jax.ShapeDtypeStruct((TS, hs[0]), jnp.bfloat16),
            jax.ShapeDtypeStruct((TS, hs[1]), jnp.bfloat16),
            jax.ShapeDtypeStruct((TS, hs[2]), jnp.bfloat16),
            jax.ShapeDtypeStruct((TS, 1), jnp.int32),
        ],
        scratch_shapes=[
            pltpu.VMEM((T, hs[0]), jnp.bfloat16),
            pltpu.VMEM((T, hs[1]), jnp.bfloat16),
            pltpu.VMEM((T, hs[2]), jnp.bfloat16),
        ],
    )(x, W1_1, W1_2, W1_3, t2, pos)

    oc1 = jnp.stack([aoff8[1, 0], cnt8[1, 0]])
    oc2 = jnp.stack([aoff8[2, 0], cnt8[2, 0]])
    oc3 = jnp.stack([aoff8[3, 0], cnt8[3, 0]])

    l1 = _tail_logprob(oc1, h1s, W2_1, ts, CUTOFFS[0], 2048)
    l2 = _tail_logprob(oc2, h2s, W2_2, ts, CUTOFFS[1], 2048)
    l3 = _tail_logprob(oc3, h3s, W2_3, ts, CUTOFFS[2], 2048)

    body = functools.partial(_head_body, n_tb=n_tb, n_tokens=T)
    out, loss = pl.pallas_call(
        body,
        grid=(n_tb,),
        in_specs=[
            pl.BlockSpec((TB, F), lambda tb: (tb, 0)),
            pl.BlockSpec(head_W.shape, lambda tb: (0, 0)),
            pl.BlockSpec((TB, 1), lambda tb: (tb, 0)),
            pl.BlockSpec((TB, 1), lambda tb: (tb, 0)),
            pl.BlockSpec((TS, 1), lambda tb: (0, 0)),
            pl.BlockSpec((TS, 1), lambda tb: (0, 0)),
            pl.BlockSpec((TS, 1), lambda tb: (0, 0)),
        ],
        out_specs=[
            pl.BlockSpec((TB, 1), lambda tb: (tb, 0)),
            pl.BlockSpec((1, 1), lambda tb: (0, 0)),
        ],
        out_shape=[
            jax.ShapeDtypeStruct((T, 1), jnp.float32),
            jax.ShapeDtypeStruct((1, 1), jnp.float32),
        ],
        scratch_shapes=[pltpu.SMEM((1,), jnp.float32)],
    )(x, head_W, t2, pos, l1, l2, l3)
    return (out.reshape(T), loss[0, 0])


# speedup vs baseline: 1.1143x; 1.1143x over previous
import functools

import jax
import jax.numpy as jnp
from jax.experimental import pallas as pl
from jax.experimental.pallas import tpu as pltpu

CUTOFFS = [2000, 10000, 50000]
SHORTLIST = 2000
NEG_INF = float("-inf")
TB = 256


def _cluster_of(t):
    return ((t >= CUTOFFS[0]).astype(jnp.int32)
            + (t >= CUTOFFS[1]).astype(jnp.int32)
            + (t >= CUTOFFS[2]).astype(jnp.int32))


def _prep1_body(t_ref, pos_ref, cnt_ref, aoff_ref, *, n_tokens):
    tb = pl.program_id(0)
    t_all = t_ref[...]
    cl_all = _cluster_of(t_all)
    cl_row = jnp.transpose(cl_all)
    row0 = tb * TB
    cl_blk = _cluster_of(t_ref[pl.ds(row0, TB), :])
    cols = jax.lax.broadcasted_iota(jnp.int32, (TB, n_tokens), 1)
    rows = row0 + jax.lax.broadcasted_iota(jnp.int32, (TB, n_tokens), 0)
    rank = jnp.sum(jnp.where((cl_row == cl_blk) & (cols < rows), 1, 0),
                   axis=1, keepdims=True)
    cidx = jax.lax.broadcasted_iota(jnp.int32, (8, n_tokens), 0)
    cnt = jnp.sum((cl_row == cidx).astype(jnp.int32), axis=1, keepdims=True)
    rup = cnt
    c_r = jax.lax.broadcasted_iota(jnp.int32, (8, 8), 0)
    c_c = jax.lax.broadcasted_iota(jnp.int32, (8, 8), 1)
    aoff = jnp.sum(jnp.where(c_c < c_r, jnp.transpose(rup), 0), axis=1,
                   keepdims=True)
    cl8 = jax.lax.broadcasted_iota(jnp.int32, (TB, 8), 1)
    my_off = jnp.sum(jnp.where(cl_blk == cl8, jnp.transpose(aoff), 0),
                     axis=1, keepdims=True)
    pos_ref[...] = my_off + rank

    @pl.when(tb == 0)
    def _counts():
        cnt_ref[...] = cnt
        aoff_ref[...] = aoff


def _prep2_body(x_ref, w11_ref, w12_ref, w13_ref, t_ref, pos_ref,
                h1_ref, h2_ref, h3_ref, ts_ref, hid1_ref, hid2_ref, hid3_ref,
                *, n_tokens):
    tb = pl.program_id(0)

    @pl.when(tb == 0)
    def _hidden():
        xb = x_ref[...].astype(jnp.bfloat16)
        for w_ref, hid_ref in ((w11_ref, hid1_ref), (w12_ref, hid2_ref),
                               (w13_ref, hid3_ref)):
            hid_ref[...] = jax.lax.dot_general(
                xb, w_ref[...].astype(jnp.bfloat16), (((1,), (1,)), ((), ())),
                preferred_element_type=jnp.float32).astype(jnp.bfloat16)

    row0 = tb * TB
    p_rows = row0 + jax.lax.broadcasted_iota(jnp.int32, (TB, n_tokens), 0)
    pos_row = jnp.transpose(pos_ref[...])
    s_blk = (pos_row == p_rows)
    s_bf = s_blk.astype(jnp.bfloat16)
    h1_ref[...] = jax.lax.dot_general(
        s_bf, hid1_ref[...], (((1,), (0,)), ((), ())),
        preferred_element_type=jnp.float32).astype(jnp.bfloat16)
    h2_ref[...] = jax.lax.dot_general(
        s_bf, hid2_ref[...], (((1,), (0,)), ((), ())),
        preferred_element_type=jnp.float32).astype(jnp.bfloat16)
    h3_ref[...] = jax.lax.dot_general(
        s_bf, hid3_ref[...], (((1,), (0,)), ((), ())),
        preferred_element_type=jnp.float32).astype(jnp.bfloat16)
    t_row = jnp.transpose(t_ref[...])
    ts_ref[...] = jnp.sum(jnp.where(s_blk, t_row, 0), axis=1, keepdims=True)


def _tail_body(oc_ref, h_ref, w2_ref, ts_ref, out_ref, m_ref, s_ref, tg_ref,
               *, low, n_cols, blk, n_blk, n_tb):
    cb = pl.program_id(0)
    off = oc_ref[0]
    cnt = oc_ref[1]

    @pl.when(cb == 0)
    def _init():
        m_ref[...] = jnp.full(m_ref.shape, NEG_INF, jnp.float32)
        s_ref[...] = jnp.zeros(s_ref.shape, jnp.float32)
        tg_ref[...] = jnp.zeros(tg_ref.shape, jnp.float32)

    def _update(masked):
        w2b = w2_ref[...].astype(jnp.bfloat16)
        for j in range(n_tb):
            row0 = j * TB
            active = (row0 < off + cnt) & (row0 + TB > off)

            @pl.when(active)
            def _step(row0=row0):
                hid = h_ref[row0:row0 + TB, :]
                logits = jax.lax.dot_general(
                    hid, w2b, (((1,), (1,)), ((), ())),
                    preferred_element_type=jnp.float32)
                cols = cb * blk + jax.lax.broadcasted_iota(
                    jnp.int32, logits.shape, 1)
                if masked:
                    lm = jnp.where(cols < n_cols, logits, NEG_INF)
                else:
                    lm = logits
                bm = jnp.max(lm, axis=1, keepdims=True)
                m_old = m_ref[row0:row0 + TB, :]
                m_new = jnp.maximum(m_old, bm)
                s_ref[row0:row0 + TB, :] = (
                    s_ref[row0:row0 + TB, :] * jnp.exp(m_old - m_new)
                    + jnp.sum(jnp.exp(lm - m_new), axis=1, keepdims=True))
                m_ref[row0:row0 + TB, :] = m_new
                rel = jnp.clip(ts_ref[row0:row0 + TB, :] - low, 0, n_cols - 1)
                tg_ref[row0:row0 + TB, :] = tg_ref[row0:row0 + TB, :] + jnp.sum(
                    jnp.where(cols == rel, lm, 0.0), axis=1, keepdims=True)

    @pl.when(cb < n_blk - 1)
    def _full_blocks():
        _update(masked=False)

    @pl.when(cb == n_blk - 1)
    def _last_block():
        _update(masked=True)
        rows = jax.lax.broadcasted_iota(jnp.int32, (m_ref.shape[0], 1), 0)
        valid = (rows >= off) & (rows < off + cnt)
        val = tg_ref[...] - m_ref[...] - jnp.log(s_ref[...])
        out_ref[...] = jnp.where(valid, val, 0.0)


def _tail_logprob(offcnt, h_sorted, w2, ts, low, blk):
    T = h_sorted.shape[0]
    n_cols, h = w2.shape
    n_blk = pl.cdiv(n_cols, blk)
    n_tb = T // TB
    body = functools.partial(_tail_body, low=low, n_cols=n_cols, blk=blk,
                             n_blk=n_blk, n_tb=n_tb)
    grid_spec = pltpu.PrefetchScalarGridSpec(
        num_scalar_prefetch=1,
        grid=(n_blk,),
        in_specs=[
            pl.BlockSpec((T, h), lambda cb, oc: (0, 0)),
            pl.BlockSpec((blk, h), lambda cb, oc: (cb, 0)),
            pl.BlockSpec((T, 1), lambda cb, oc: (0, 0)),
        ],
        out_specs=pl.BlockSpec((T, 1), lambda cb, oc: (0, 0)),
        scratch_shapes=[
            pltpu.VMEM((T, 1), jnp.float32),
            pltpu.VMEM((T, 1), jnp.float32),
            pltpu.VMEM((T, 1), jnp.float32),
        ],
    )
    return pl.pallas_call(
        body,
        grid_spec=grid_spec,
        out_shape=jax.ShapeDtypeStruct((T, 1), jnp.float32),
    )(offcnt, h_sorted, w2, ts)


def _head_body(x_ref, hw_ref, t_ref, pos_ref, l1_ref, l2_ref, l3_ref, out_ref,
               loss_ref, acc_ref, *, n_tb, n_tokens):
    tb = pl.program_id(0)
    logits = jax.lax.dot_general(
        x_ref[...].astype(jnp.bfloat16), hw_ref[...].astype(jnp.bfloat16),
        (((1,), (1,)), ((), ())),
        preferred_element_type=jnp.float32)
    cols = jax.lax.broadcasted_iota(jnp.int32, logits.shape, 1)
    m = jnp.max(logits, axis=1, keepdims=True)
    s = jnp.sum(jnp.exp(logits - m), axis=1, keepdims=True)
    t = t_ref[...]
    cl = _cluster_of(t)
    gidx = jnp.where(cl == 0, t, SHORTLIST + cl - 1)
    ht = (jnp.sum(jnp.where(cols == gidx, logits, 0.0), axis=1, keepdims=True)
          - m - jnp.log(s))
    lsum = l1_ref[...] + l2_ref[...] + l3_ref[...]
    p_rows = jax.lax.broadcasted_iota(jnp.int32, (lsum.shape[0], TB), 0)
    s_blk = (jnp.transpose(pos_ref[...]) == p_rows).astype(jnp.float32)
    local = jax.lax.dot_general(
        s_blk, lsum, (((0,), (0,)), ((), ())),
        preferred_element_type=jnp.float32)
    out = local + ht
    out_ref[...] = out

    @pl.when(tb == 0)
    def _z():
        acc_ref[0] = 0.0

    acc_ref[0] += jnp.sum(-out) / n_tokens

    @pl.when(tb == n_tb - 1)
    def _w():
        loss_ref[...] = jnp.full((1, 1), acc_ref[0], jnp.float32)


def kernel(myinput, target, head_W, W1_1, W2_1, W1_2, W2_2, W1_3, W2_3):
    x = myinput
    T, F = x.shape
    t2 = target.astype(jnp.int32).reshape(T, 1)
    n_tb = T // TB

    pos, cnt8, aoff8 = pl.pallas_call(
        functools.partial(_prep1_body, n_tokens=T),
        grid=(n_tb,),
        in_specs=[pl.BlockSpec((T, 1), lambda tb: (0, 0))],
        out_specs=[
            pl.BlockSpec((TB, 1), lambda tb: (tb, 0)),
            pl.BlockSpec((8, 1), lambda tb: (0, 0)),
            pl.BlockSpec((8, 1), lambda tb: (0, 0)),
        ],
        out_shape=[
            jax.ShapeDtypeStruct((T, 1), jnp.int32),
            jax.ShapeDtypeStruct((8, 1), jnp.int32),
            jax.ShapeDtypeStruct((8, 1), jnp.int32),
        ],
    )(t2)

    hs = [W1_1.shape[0], W1_2.shape[0], W1_3.shape[0]]
    h1s, h2s, h3s, ts = pl.pallas_call(
        functools.partial(_prep2_body, n_tokens=T),
        grid=(n_tb,),
        in_specs=[
            pl.BlockSpec((T, F), lambda tb: (0, 0)),
            pl.BlockSpec(W1_1.shape, lambda tb: (0, 0)),
            pl.BlockSpec(W1_2.shape, lambda tb: (0, 0)),
            pl.BlockSpec(W1_3.shape, lambda tb: (0, 0)),
            pl.BlockSpec((T, 1), lambda tb: (0, 0)),
            pl.BlockSpec((T, 1), lambda tb: (0, 0)),
        ],
        out_specs=[
            pl.BlockSpec((TB, hs[0]), lambda tb: (tb, 0)),
            pl.BlockSpec((TB, hs[1]), lambda tb: (tb, 0)),
            pl.BlockSpec((TB, hs[2]), lambda tb: (tb, 0)),
            pl.BlockSpec((TB, 1), lambda tb: (tb, 0)),
        ],
        out_shape=[
            jax.ShapeDtypeStruct((T, hs[0]), jnp.bfloat16),
            jax.ShapeDtypeStruct((T, hs[1]), jnp.bfloat16),
            jax.ShapeDtypeStruct((T, hs[2]), jnp.bfloat16),
            jax.ShapeDtypeStruct((T, 1), jnp.int32),
        ],
        scratch_shapes=[
            pltpu.VMEM((T, hs[0]), jnp.bfloat16),
            pltpu.VMEM((T, hs[1]), jnp.bfloat16),
            pltpu.VMEM((T, hs[2]), jnp.bfloat16),
        ],
    )(x, W1_1, W1_2, W1_3, t2, pos)

    oc1 = jnp.stack([aoff8[1, 0], cnt8[1, 0]])
    oc2 = jnp.stack([aoff8[2, 0], cnt8[2, 0]])
    oc3 = jnp.stack([aoff8[3, 0], cnt8[3, 0]])

    l1 = _tail_logprob(oc1, h1s, W2_1, ts, CUTOFFS[0], 2048)
    l2 = _tail_logprob(oc2, h2s, W2_2, ts, CUTOFFS[1], 2048)
    l3 = _tail_logprob(oc3, h3s, W2_3, ts, CUTOFFS[2], 2048)

    body = functools.partial(_head_body, n_tb=n_tb, n_tokens=T)
    out, loss = pl.pallas_call(
        body,
        grid=(n_tb,),
        in_specs=[
            pl.BlockSpec((TB, F), lambda tb: (tb, 0)),
            pl.BlockSpec(head_W.shape, lambda tb: (0, 0)),
            pl.BlockSpec((TB, 1), lambda tb: (tb, 0)),
            pl.BlockSpec((TB, 1), lambda tb: (tb, 0)),
            pl.BlockSpec((T, 1), lambda tb: (0, 0)),
            pl.BlockSpec((T, 1), lambda tb: (0, 0)),
            pl.BlockSpec((T, 1), lambda tb: (0, 0)),
        ],
        out_specs=[
            pl.BlockSpec((TB, 1), lambda tb: (tb, 0)),
            pl.BlockSpec((1, 1), lambda tb: (0, 0)),
        ],
        out_shape=[
            jax.ShapeDtypeStruct((T, 1), jnp.float32),
            jax.ShapeDtypeStruct((1, 1), jnp.float32),
        ],
        scratch_shapes=[pltpu.SMEM((1,), jnp.float32)],
    )(x, head_W, t2, pos, l1, l2, l3)
    return (out.reshape(T), loss[0, 0])
